# baseline (speedup 1.0000x reference)
import functools

import jax
import jax.numpy as jnp
from jax import lax
from jax.experimental import pallas as pl
from jax.experimental.pallas import tpu as pltpu
from jax.experimental.pallas import tpu_sc as plsc

NN = 10000
EE = 320000
BG = 256
FIN = 128
DD = 256
HALF = 128
NC, NS = 2, 16
IW = 80
EW = 128
EPW = 162
EP = NS * EPW * EW
NR = 3
NI = 6
DUMP = NN
CP = 632
CPL = NN - (NS - 1) * CP
IDL = 10240
ICT = IDL // IW // NS
OCT = NN // IW
OCL = OCT - (NS - 1) * ICT
NGP = 2176

_MESH = plsc.VectorSubcoreMesh(
    core_axis_name="c", subcore_axis_name="s", num_cores=NC, num_subcores=NS)


def _deg_body(dst1, idn, deg_out, ones_v, idxb, idxe, rows16, degacc, sem):
    c = lax.axis_index("c")
    s = lax.axis_index("s")

    def fill(i, carry):
        ones_v[i] = jnp.ones((16,), jnp.float32)
        return carry
    lax.fori_loop(0, EW, fill, 0)

    @pl.when(c == 0)
    def _():
        def ik(k, carry):
            pltpu.sync_copy(idn.at[pl.ds((s * ICT + k) * IW, IW)], idxb)
            pltpu.sync_copy(ones_v.at[pl.ds(0, IW)], degacc.at[idxb])
            return carry
        lax.fori_loop(0, ICT, ik, 0)
    plsc.subcore_barrier()

    @pl.when(c == 0)
    def _():
        def step(g, carry):
            pltpu.sync_copy(dst1.at[pl.ds((s * EPW + g) * EW, EW)], idxe)
            pltpu.sync_copy(ones_v, degacc.at[idxe], add=True)
            return carry
        lax.fori_loop(0, EPW, step, 0)
    plsc.subcore_barrier()

    @pl.when(c == 0)
    def _():
        nk = jnp.where(s < NS - 1, ICT, OCL)

        def ok(k, carry):
            row = (s * ICT + k) * IW
            pltpu.sync_copy(idn.at[pl.ds(row, IW)], idxb)
            pltpu.async_copy(degacc.at[idxb], rows16, sem).wait()
            pltpu.sync_copy(rows16, deg_out.at[pl.ds(row, IW)])
            return carry
        lax.fori_loop(0, nk, ok, 0)


@functools.partial(
    pl.kernel,
    out_type=jax.ShapeDtypeStruct((NN, 16), jnp.float32),
    mesh=_MESH,
    scratch_types=[
        pltpu.VMEM((EW, 16), jnp.float32),
        pltpu.VMEM((IW,), jnp.int32),
        pltpu.VMEM((EW,), jnp.int32),
        pltpu.VMEM((IW, 16), jnp.float32),
        pltpu.VMEM_SHARED((NN + 8, 16), jnp.float32),
        pltpu.SemaphoreType.DMA,
    ],
)
def _deg(dst1, idn, deg_out, ones_v, idxb, idxe, rows16, degacc, sem):
    _deg_body(dst1, idn, deg_out, ones_v, idxb, idxe, rows16, degacc, sem)


def _agg_body(hpre, srcg2, dst2, idn, out, acc, srcb0, srcb1, dstb0, dstb1,
              idxb, rows0, rows1, sem, gs0, gs1):
    c = lax.axis_index("c")
    s = lax.axis_index("s")

    srcb = (srcb0, srcb1)
    dstb = (dstb0, dstb1)
    rows = (rows0, rows1)
    gs = (gs0, gs1)
    ir = rows0.at[pl.ds(0, IW)]

    def ik(k, carry):
        kk = s * ICT + k
        off = jnp.where(kk < OCT, kk * IW, 0)
        pltpu.sync_copy(idn.at[pl.ds(kk * IW, IW)], idxb)
        pltpu.sync_copy(hpre.at[pl.ds(c * NN + off, IW)], ir)
        pltpu.sync_copy(ir, acc.at[idxb])
        return carry
    lax.fori_loop(0, ICT, ik, 0)
    plsc.subcore_barrier()
    soff = (c * NS + s) * EPW * EW
    doff = s * EPW * EW

    def fetch(g, b):
        pltpu.sync_copy(srcg2.at[pl.ds(soff + g * EW, EW)], srcb[b])
        pltpu.sync_copy(dst2.at[pl.ds(doff + g * EW, EW)], dstb[b])
        pltpu.async_copy(hpre.at[srcb[b]], rows[b], gs[b])

    def drain(b):
        pltpu.make_async_copy(hpre.at[srcb[b]], rows[b], gs[b]).wait()

    fetch(0, 0)
    fetch(1, 1)

    def step(h, carry):
        for b in range(2):
            g = 2 * h + b
            drain(b)
            pltpu.sync_copy(rows[b], acc.at[dstb[b]], add=True)
            nxt = jnp.where(g + 2 < EPW, g + 2, 0)
            fetch(nxt, b)
        return carry
    lax.fori_loop(0, EPW // 2, step, 0)
    drain(0)
    drain(1)
    plsc.subcore_barrier()

    nk = jnp.where(s < NS - 1, ICT, OCL)

    def ok(k, carry):
        row = (s * ICT + k) * IW
        pltpu.sync_copy(idn.at[pl.ds(row, IW)], idxb)
        pltpu.async_copy(acc.at[idxb], ir, sem).wait()
        pltpu.sync_copy(ir, out.at[pl.ds(c * NN + row, IW)])
        return carry
    lax.fori_loop(0, nk, ok, 0)


@functools.partial(
    pl.kernel,
    out_type=jax.ShapeDtypeStruct((2 * NN, HALF), jnp.float32),
    mesh=_MESH,
    scratch_types=[
        pltpu.VMEM_SHARED((NN + 8, HALF), jnp.float32),
        pltpu.VMEM((EW,), jnp.int32),
        pltpu.VMEM((EW,), jnp.int32),
        pltpu.VMEM((EW,), jnp.int32),
        pltpu.VMEM((EW,), jnp.int32),
        pltpu.VMEM((IW,), jnp.int32),
        pltpu.VMEM((EW, HALF), jnp.float32),
        pltpu.VMEM((EW, HALF), jnp.float32),
        pltpu.SemaphoreType.DMA,
        pltpu.SemaphoreType.DMA,
        pltpu.SemaphoreType.DMA,
    ],
)
def _agg(hpre, srcg2, dst2, idn, out, acc, srcb0, srcb1, dstb0, dstb1,
         idxb, rows0, rows1, sem, gs0, gs1):
    _agg_body(hpre, srcg2, dst2, idn, out, acc, srcb0, srcb1, dstb0, dstb1,
              idxb, rows0, rows1, sem, gs0, gs1)


RB = 1000
NBLK = NN // RB


def _tc0_body(x_ref, deg_ref, w_ref, out_ref):
    dinv = lax.rsqrt(deg_ref[:, 0:1])
    h = jnp.dot(x_ref[:], w_ref[:], preferred_element_type=jnp.float32)
    h = h * dinv
    out_ref[0] = h[:, :HALF]
    out_ref[1] = h[:, HALF:]


def _tc0(x, deg16, w):
    return pl.pallas_call(
        _tc0_body,
        grid=(NBLK,),
        in_specs=[
            pl.BlockSpec((RB, FIN), lambda i: (i, 0)),
            pl.BlockSpec((RB, 16), lambda i: (i, 0)),
            pl.BlockSpec((FIN, DD), lambda i: (0, 0)),
        ],
        out_specs=pl.BlockSpec((2, RB, HALF), lambda i: (0, i, 0)),
        out_shape=jax.ShapeDtypeStruct((2, NN, HALF), jnp.float32),
    )(x, deg16, w)


def _tcm_body(a_ref, deg_ref, w_ref, bprev_ref, out_ref):
    dinv = lax.rsqrt(deg_ref[:, 0:1])
    h = jnp.concatenate([a_ref[0], a_ref[1]], axis=1)
    h = h * dinv + bprev_ref[:]
    h = jnp.where(h > 0, h, 0.01 * h)
    h2 = jnp.dot(h, w_ref[:], preferred_element_type=jnp.float32) * dinv
    out_ref[0] = h2[:, :HALF]
    out_ref[1] = h2[:, HALF:]


def _tcm(a, deg16, w, bprev):
    return pl.pallas_call(
        _tcm_body,
        grid=(NBLK,),
        in_specs=[
            pl.BlockSpec((2, RB, HALF), lambda i: (0, i, 0)),
            pl.BlockSpec((RB, 16), lambda i: (i, 0)),
            pl.BlockSpec((DD, DD), lambda i: (0, 0)),
            pl.BlockSpec((1, DD), lambda i: (0, 0)),
        ],
        out_specs=pl.BlockSpec((2, RB, HALF), lambda i: (0, i, 0)),
        out_shape=jax.ShapeDtypeStruct((2, NN, HALF), jnp.float32),
    )(a, deg16, w, bprev)


def _gene_body(exp_ref, wg_ref, bg_ref, ww_ref, bw_ref, g_ref):
    e = jnp.dot(exp_ref[:], wg_ref[:], preferred_element_type=jnp.float32)
    e = jnp.maximum(e + bg_ref[:], 0.0)
    g_ref[:] = jnp.dot(e, ww_ref[:], preferred_element_type=jnp.float32) \
        + bw_ref[:]


def _gene(expp, wgp, bg, ww, bw):
    return pl.pallas_call(
        _gene_body,
        out_shape=jax.ShapeDtypeStruct((BG, DD), jnp.float32),
    )(expp, wgp, bg, ww, bw)


def _fin_body(a_ref, deg_ref, batch_ref, g_ref, b2_ref, wf1_ref, bf1_ref,
              wf2_ref, bf2_ref, out_ref, seg_acc, cnt_acc):
    i = pl.program_id(0)

    @pl.when(i == 0)
    def _():
        seg_acc[:] = jnp.zeros_like(seg_acc)
        cnt_acc[:] = jnp.zeros_like(cnt_acc)

    dinv = lax.rsqrt(deg_ref[:, 0:1])
    h3 = jnp.concatenate([a_ref[0], a_ref[1]], axis=1) * dinv + b2_ref[:]
    b = batch_ref[:][0, 0]
    oh = (lax.broadcasted_iota(jnp.int32, (BG, RB), 0)
          == b[None, :]).astype(jnp.float32)
    seg_acc[:] += jnp.dot(oh, h3, preferred_element_type=jnp.float32)
    cnt_acc[:] += jnp.sum(oh, axis=1, keepdims=True)

    @pl.when(i == NBLK - 1)
    def _():
        cnt = cnt_acc[:]
        pooled = seg_acc[:] / jnp.maximum(cnt, 1.0) \
            + g_ref[:] * (cnt > 0).astype(jnp.float32)
        z = jnp.dot(pooled, wf1_ref[:], preferred_element_type=jnp.float32)
        z = jnp.maximum(z + bf1_ref[:], 0.0)
        out_ref[:] = jnp.dot(z, wf2_ref[:],
                             preferred_element_type=jnp.float32) + bf2_ref[:]


def _fin(a, deg16, batch, g, b2, wf1, bf1, wf2, bf2):
    return pl.pallas_call(
        _fin_body,
        grid=(NBLK,),
        in_specs=[
            pl.BlockSpec((2, RB, HALF), lambda i: (0, i, 0)),
            pl.BlockSpec((RB, 16), lambda i: (i, 0)),
            pl.BlockSpec((1, 1, RB), lambda i: (i, 0, 0)),
            pl.BlockSpec((BG, DD), lambda i: (0, 0)),
            pl.BlockSpec((1, DD), lambda i: (0, 0)),
            pl.BlockSpec((DD, DD), lambda i: (0, 0)),
            pl.BlockSpec((1, DD), lambda i: (0, 0)),
            pl.BlockSpec((DD, 1), lambda i: (0, 0)),
            pl.BlockSpec((1, 1), lambda i: (0, 0)),
        ],
        out_specs=pl.BlockSpec((BG, 1), lambda i: (0, 0)),
        out_shape=jax.ShapeDtypeStruct((BG, 1), jnp.float32),
        scratch_shapes=[
            pltpu.VMEM((BG, DD), jnp.float32),
            pltpu.VMEM((BG, 1), jnp.float32),
        ],
    )(a, deg16, batch, g, b2, wf1, bf1, wf2, bf2)


def kernel(exp, x, edge_index, edge_attr, batch, W_gene, b_gene, W_wg, b_wg,
           W_g0, b_g0, W_g1, b_g1, W_g2, b_g2, W_fc1, b_fc1, W_fc2, b_fc2):
    src = edge_index[0].astype(jnp.int32)
    dst = edge_index[1].astype(jnp.int32)
    srcp = jnp.concatenate([src, jnp.zeros((EP - EE,), jnp.int32)])
    dstp = jnp.concatenate([dst, jnp.full((EP - EE,), DUMP, jnp.int32)])
    dst2 = dstp
    srcg2 = jnp.concatenate([srcp, srcp + NN])

    idn = jnp.concatenate([jnp.arange(NN, dtype=jnp.int32),
                           jnp.full((IDL - NN,), DUMP, jnp.int32)])
    deg16 = _deg(dst2, idn)

    def _do_agg(h2):
        return _agg(h2, srcg2, dst2, idn)

    h = _tc0(x, deg16, W_g0)
    a = _do_agg(h.reshape(2 * NN, HALF))
    h = _tcm(a.reshape(2, NN, HALF), deg16, W_g1, b_g0.reshape(1, DD))
    a = _do_agg(h.reshape(2 * NN, HALF))
    h = _tcm(a.reshape(2, NN, HALF), deg16, W_g2, b_g1.reshape(1, DD))
    a = _do_agg(h.reshape(2 * NN, HALF))

    expp = jnp.pad(exp, ((0, 0), (0, NGP - exp.shape[1])))
    wgp = jnp.pad(W_gene, ((0, NGP - W_gene.shape[0]), (0, 0)))
    g = _gene(expp, wgp, b_gene.reshape(1, -1), W_wg, b_wg.reshape(1, DD))

    return _fin(a.reshape(2, NN, HALF), deg16,
                batch.astype(jnp.int32).reshape(NBLK, 1, RB), g,
                b_g2.reshape(1, DD), W_fc1, b_fc1.reshape(1, DD), W_fc2,
                b_fc2.reshape(1, 1))


# speedup vs baseline: 1.4123x; 1.4123x over previous
import functools

import jax
import jax.numpy as jnp
from jax import lax
from jax.experimental import pallas as pl
from jax.experimental.pallas import tpu as pltpu
from jax.experimental.pallas import tpu_sc as plsc

NN = 10000
EE = 320000
BG = 256
FIN = 128
DD = 256
HALF = 128
NC, NS = 2, 16
IW = 80
EW = 128
EPW = 160
EP = NS * EPW * EW
NR = 3
NI = 6
DUMP = NN
CP = 632
CPL = NN - (NS - 1) * CP
IDL = 10240
ICT = IDL // IW // NS
OCT = NN // IW
OCL = OCT - (NS - 1) * ICT
NGP = 2176

_MESH = plsc.VectorSubcoreMesh(
    core_axis_name="c", subcore_axis_name="s", num_cores=NC, num_subcores=NS)


def _deg_body(dst1, idn, deg_out, ones_v, idxb, idxe, rows16, degacc, sem):
    c = lax.axis_index("c")
    s = lax.axis_index("s")

    def fill(i, carry):
        ones_v[i] = jnp.ones((16,), jnp.float32)
        return carry
    lax.fori_loop(0, EW, fill, 0)

    @pl.when(c == 0)
    def _():
        def ik(k, carry):
            pltpu.sync_copy(idn.at[pl.ds((s * ICT + k) * IW, IW)], idxb)
            pltpu.sync_copy(ones_v.at[pl.ds(0, IW)], degacc.at[idxb])
            return carry
        lax.fori_loop(0, ICT, ik, 0)
    plsc.subcore_barrier()

    @pl.when(c == 0)
    def _():
        def step(g, carry):
            pltpu.sync_copy(dst1.at[pl.ds((s * EPW + g) * IW, IW)], idxb)
            pltpu.sync_copy(ones_v.at[pl.ds(0, IW)],
                            degacc.at[idxb], add=True)
            return carry
        lax.fori_loop(0, EPW, step, 0)
    plsc.subcore_barrier()

    @pl.when(c == 0)
    def _():
        nk = jnp.where(s < NS - 1, ICT, OCL)

        def ok(k, carry):
            row = (s * ICT + k) * IW
            pltpu.sync_copy(idn.at[pl.ds(row, IW)], idxb)
            pltpu.async_copy(degacc.at[idxb], rows16, sem).wait()
            pltpu.sync_copy(rows16, deg_out.at[pl.ds(row, IW)])
            return carry
        lax.fori_loop(0, nk, ok, 0)


@functools.partial(
    pl.kernel,
    out_type=jax.ShapeDtypeStruct((NN, 16), jnp.float32),
    mesh=_MESH,
    scratch_types=[
        pltpu.VMEM((EW, 16), jnp.float32),
        pltpu.VMEM((IW,), jnp.int32),
        pltpu.VMEM((EW,), jnp.int32),
        pltpu.VMEM((IW, 16), jnp.float32),
        pltpu.VMEM_SHARED((NN + 8, 16), jnp.float32),
        pltpu.SemaphoreType.DMA,
    ],
)
def _deg(dst1, idn, deg_out, ones_v, idxb, idxe, rows16, degacc, sem):
    _deg_body(dst1, idn, deg_out, ones_v, idxb, idxe, rows16, degacc, sem)


def _agg_body(hpre, srcg2, dst2, idn, out, acc, srcb0, srcb1, dstb0, dstb1,
              idxb, rows0, rows1, sem, gs0, gs1):
    c = lax.axis_index("c")
    s = lax.axis_index("s")

    srcb = (srcb0, srcb1)
    dstb = (dstb0, dstb1)
    rows = (rows0, rows1)
    gs = (gs0, gs1)
    ir = rows0.at[pl.ds(0, IW)]

    def ik(k, carry):
        kk = s * ICT + k
        off = jnp.where(kk < OCT, kk * IW, 0)
        pltpu.sync_copy(idn.at[pl.ds(kk * IW, IW)], idxb)
        pltpu.sync_copy(hpre.at[pl.ds(c * NN + off, IW)], ir)
        pltpu.sync_copy(ir, acc.at[idxb])
        return carry
    lax.fori_loop(0, ICT, ik, 0)
    plsc.subcore_barrier()
    soff = (c * NS + s) * EPW * EW
    doff = s * EPW * EW

    def fetch(g, b):
        pltpu.sync_copy(srcg2.at[pl.ds(soff + g * EW, EW)], srcb[b])
        pltpu.sync_copy(dst2.at[pl.ds(doff + g * EW, EW)], dstb[b])
        pltpu.async_copy(hpre.at[srcb[b]], rows[b], gs[b])

    def drain(b):
        pltpu.make_async_copy(hpre.at[srcb[b]], rows[b], gs[b]).wait()

    fetch(0, 0)
    fetch(1, 1)

    def step(h, carry):
        for b in range(2):
            g = 2 * h + b
            drain(b)
            pltpu.sync_copy(rows[b], acc.at[dstb[b]], add=True)
            nxt = jnp.where(g + 2 < EPW, g + 2, 0)
            fetch(nxt, b)
        return carry
    lax.fori_loop(0, EPW // 2, step, 0)
    drain(0)
    drain(1)
    plsc.subcore_barrier()

    nk = jnp.where(s < NS - 1, ICT, OCL)

    def ok(k, carry):
        row = (s * ICT + k) * IW
        pltpu.sync_copy(idn.at[pl.ds(row, IW)], idxb)
        pltpu.async_copy(acc.at[idxb], ir, sem).wait()
        pltpu.sync_copy(ir, out.at[pl.ds(c * NN + row, IW)])
        return carry
    lax.fori_loop(0, nk, ok, 0)


@functools.partial(
    pl.kernel,
    out_type=jax.ShapeDtypeStruct((2 * NN, HALF), jnp.float32),
    mesh=_MESH,
    scratch_types=[
        pltpu.VMEM_SHARED((NN + 8, HALF), jnp.float32),
        pltpu.VMEM((EW,), jnp.int32),
        pltpu.VMEM((EW,), jnp.int32),
        pltpu.VMEM((EW,), jnp.int32),
        pltpu.VMEM((EW,), jnp.int32),
        pltpu.VMEM((IW,), jnp.int32),
        pltpu.VMEM((EW, HALF), jnp.float32),
        pltpu.VMEM((EW, HALF), jnp.float32),
        pltpu.SemaphoreType.DMA,
        pltpu.SemaphoreType.DMA,
        pltpu.SemaphoreType.DMA,
    ],
)
def _agg(hpre, srcg2, dst2, idn, out, acc, srcb0, srcb1, dstb0, dstb1,
         idxb, rows0, rows1, sem, gs0, gs1):
    _agg_body(hpre, srcg2, dst2, idn, out, acc, srcb0, srcb1, dstb0, dstb1,
              idxb, rows0, rows1, sem, gs0, gs1)


RB = 1000
NBLK = NN // RB


def _tc0_body(x_ref, deg_ref, w_ref, out_ref):
    dinv = lax.rsqrt(deg_ref[:, 0:1])
    h = jnp.dot(x_ref[:], w_ref[:], preferred_element_type=jnp.float32)
    h = h * dinv
    out_ref[0] = h[:, :HALF]
    out_ref[1] = h[:, HALF:]


def _tc0(x, deg16, w):
    return pl.pallas_call(
        _tc0_body,
        grid=(NBLK,),
        in_specs=[
            pl.BlockSpec((RB, FIN), lambda i: (i, 0)),
            pl.BlockSpec((RB, 16), lambda i: (i, 0)),
            pl.BlockSpec((FIN, DD), lambda i: (0, 0)),
        ],
        out_specs=pl.BlockSpec((2, RB, HALF), lambda i: (0, i, 0)),
        out_shape=jax.ShapeDtypeStruct((2, NN, HALF), jnp.float32),
    )(x, deg16, w)


def _tcm_body(a_ref, deg_ref, w_ref, bprev_ref, out_ref):
    dinv = lax.rsqrt(deg_ref[:, 0:1])
    h = jnp.concatenate([a_ref[0], a_ref[1]], axis=1)
    h = h * dinv + bprev_ref[:]
    h = jnp.where(h > 0, h, 0.01 * h)
    h2 = jnp.dot(h, w_ref[:], preferred_element_type=jnp.float32) * dinv
    out_ref[0] = h2[:, :HALF]
    out_ref[1] = h2[:, HALF:]


def _tcm(a, deg16, w, bprev):
    return pl.pallas_call(
        _tcm_body,
        grid=(NBLK,),
        in_specs=[
            pl.BlockSpec((2, RB, HALF), lambda i: (0, i, 0)),
            pl.BlockSpec((RB, 16), lambda i: (i, 0)),
            pl.BlockSpec((DD, DD), lambda i: (0, 0)),
            pl.BlockSpec((1, DD), lambda i: (0, 0)),
        ],
        out_specs=pl.BlockSpec((2, RB, HALF), lambda i: (0, i, 0)),
        out_shape=jax.ShapeDtypeStruct((2, NN, HALF), jnp.float32),
    )(a, deg16, w, bprev)


def _gene_body(exp_ref, wg_ref, bg_ref, ww_ref, bw_ref, g_ref):
    e = jnp.dot(exp_ref[:], wg_ref[:], preferred_element_type=jnp.float32)
    e = jnp.maximum(e + bg_ref[:], 0.0)
    g_ref[:] = jnp.dot(e, ww_ref[:], preferred_element_type=jnp.float32) \
        + bw_ref[:]


def _gene(expp, wgp, bg, ww, bw):
    return pl.pallas_call(
        _gene_body,
        out_shape=jax.ShapeDtypeStruct((BG, DD), jnp.float32),
    )(expp, wgp, bg, ww, bw)


def _fin_body(a_ref, deg_ref, batch_ref, g_ref, b2_ref, wf1_ref, bf1_ref,
              wf2_ref, bf2_ref, out_ref, seg_acc, cnt_acc):
    i = pl.program_id(0)

    @pl.when(i == 0)
    def _():
        seg_acc[:] = jnp.zeros_like(seg_acc)
        cnt_acc[:] = jnp.zeros_like(cnt_acc)

    dinv = lax.rsqrt(deg_ref[:, 0:1])
    h3 = jnp.concatenate([a_ref[0], a_ref[1]], axis=1) * dinv + b2_ref[:]
    b = batch_ref[:][0, 0]
    oh = (lax.broadcasted_iota(jnp.int32, (BG, RB), 0)
          == b[None, :]).astype(jnp.float32)
    seg_acc[:] += jnp.dot(oh, h3, preferred_element_type=jnp.float32)
    cnt_acc[:] += jnp.sum(oh, axis=1, keepdims=True)

    @pl.when(i == NBLK - 1)
    def _():
        cnt = cnt_acc[:]
        pooled = seg_acc[:] / jnp.maximum(cnt, 1.0) \
            + g_ref[:] * (cnt > 0).astype(jnp.float32)
        z = jnp.dot(pooled, wf1_ref[:], preferred_element_type=jnp.float32)
        z = jnp.maximum(z + bf1_ref[:], 0.0)
        out_ref[:] = jnp.dot(z, wf2_ref[:],
                             preferred_element_type=jnp.float32) + bf2_ref[:]


def _fin(a, deg16, batch, g, b2, wf1, bf1, wf2, bf2):
    return pl.pallas_call(
        _fin_body,
        grid=(NBLK,),
        in_specs=[
            pl.BlockSpec((2, RB, HALF), lambda i: (0, i, 0)),
            pl.BlockSpec((RB, 16), lambda i: (i, 0)),
            pl.BlockSpec((1, 1, RB), lambda i: (i, 0, 0)),
            pl.BlockSpec((BG, DD), lambda i: (0, 0)),
            pl.BlockSpec((1, DD), lambda i: (0, 0)),
            pl.BlockSpec((DD, DD), lambda i: (0, 0)),
            pl.BlockSpec((1, DD), lambda i: (0, 0)),
            pl.BlockSpec((DD, 1), lambda i: (0, 0)),
            pl.BlockSpec((1, 1), lambda i: (0, 0)),
        ],
        out_specs=pl.BlockSpec((BG, 1), lambda i: (0, 0)),
        out_shape=jax.ShapeDtypeStruct((BG, 1), jnp.float32),
        scratch_shapes=[
            pltpu.VMEM((BG, DD), jnp.float32),
            pltpu.VMEM((BG, 1), jnp.float32),
        ],
    )(a, deg16, batch, g, b2, wf1, bf1, wf2, bf2)


def kernel(exp, x, edge_index, edge_attr, batch, W_gene, b_gene, W_wg, b_wg,
           W_g0, b_g0, W_g1, b_g1, W_g2, b_g2, W_fc1, b_fc1, W_fc2, b_fc2):
    src = edge_index[0].astype(jnp.int32)
    dst = edge_index[1].astype(jnp.int32)
    srcp = jnp.concatenate([src, jnp.zeros((EP - EE,), jnp.int32)])
    dstp = jnp.concatenate([dst, jnp.full((EP - EE,), DUMP, jnp.int32)])
    dst2 = dstp
    srcg2 = jnp.concatenate([srcp, srcp + NN])

    idn = jnp.concatenate([jnp.arange(NN, dtype=jnp.int32),
                           jnp.full((IDL - NN,), DUMP, jnp.int32)])
    deg16 = _deg(dst2, idn)

    def _do_agg(h2):
        return _agg(h2, srcg2, dst2, idn)

    h = _tc0(x, deg16, W_g0)
    a = _do_agg(h.reshape(2 * NN, HALF))
    h = _tcm(a.reshape(2, NN, HALF), deg16, W_g1, b_g0.reshape(1, DD))
    a = _do_agg(h.reshape(2 * NN, HALF))
    h = _tcm(a.reshape(2, NN, HALF), deg16, W_g2, b_g1.reshape(1, DD))
    a = _do_agg(h.reshape(2 * NN, HALF))

    expp = jnp.pad(exp, ((0, 0), (0, NGP - exp.shape[1])))
    wgp = jnp.pad(W_gene, ((0, NGP - W_gene.shape[0]), (0, 0)))
    g = _gene(expp, wgp, b_gene.reshape(1, -1), W_wg, b_wg.reshape(1, DD))

    return _fin(a.reshape(2, NN, HALF), deg16,
                batch.astype(jnp.int32).reshape(NBLK, 1, RB), g,
                b_g2.reshape(1, DD), W_fc1, b_fc1.reshape(1, DD), W_fc2,
                b_fc2.reshape(1, 1))


# speedup vs baseline: 1.4139x; 1.0012x over previous
import functools

import jax
import jax.numpy as jnp
from jax import lax
from jax.experimental import pallas as pl
from jax.experimental.pallas import tpu as pltpu
from jax.experimental.pallas import tpu_sc as plsc

NN = 10000
EE = 320000
BG = 256
FIN = 128
DD = 256
HALF = 128
NC, NS = 2, 16
IW = 80
EW = 128
EPW = 160
EP = NS * EPW * EW
DUMP = NN
IDL = 10240
ICT = IDL // IW // NS
OCT = NN // IW
OCL = OCT - (NS - 1) * ICT
NGP = 2176

_MESH = plsc.VectorSubcoreMesh(
    core_axis_name="c", subcore_axis_name="s", num_cores=NC, num_subcores=NS)


def _deg_body(dst1, idn, deg_out, ones_v, idxb, rows16, degacc, sem):
    c = lax.axis_index("c")
    s = lax.axis_index("s")

    def fill(i, carry):
        ones_v[i] = jnp.ones((16,), jnp.float32)
        return carry
    lax.fori_loop(0, EW, fill, 0)

    @pl.when(c == 0)
    def _():
        def ik(k, carry):
            pltpu.sync_copy(idn.at[pl.ds((s * ICT + k) * IW, IW)], idxb)
            pltpu.sync_copy(ones_v.at[pl.ds(0, IW)], degacc.at[idxb])
            return carry
        lax.fori_loop(0, ICT, ik, 0)
    plsc.subcore_barrier()

    @pl.when(c == 0)
    def _():
        def step(g, carry):
            pltpu.sync_copy(dst1.at[pl.ds((s * EPW + g) * IW, IW)], idxb)
            pltpu.sync_copy(ones_v.at[pl.ds(0, IW)],
                            degacc.at[idxb], add=True)
            return carry
        lax.fori_loop(0, EPW, step, 0)
    plsc.subcore_barrier()

    @pl.when(c == 0)
    def _():
        nk = jnp.where(s < NS - 1, ICT, OCL)

        def ok(k, carry):
            row = (s * ICT + k) * IW
            pltpu.sync_copy(idn.at[pl.ds(row, IW)], idxb)
            pltpu.async_copy(degacc.at[idxb], rows16, sem).wait()
            pltpu.sync_copy(rows16, deg_out.at[pl.ds(row, IW)])
            return carry
        lax.fori_loop(0, nk, ok, 0)


@functools.partial(
    pl.kernel,
    out_type=jax.ShapeDtypeStruct((NN, 16), jnp.float32),
    mesh=_MESH,
    scratch_types=[
        pltpu.VMEM((EW, 16), jnp.float32),
        pltpu.VMEM((IW,), jnp.int32),
        pltpu.VMEM((IW, 16), jnp.float32),
        pltpu.VMEM_SHARED((NN + 8, 16), jnp.float32),
        pltpu.SemaphoreType.DMA,
    ],
)
def _deg(dst1, idn, deg_out, ones_v, idxb, rows16, degacc, sem):
    _deg_body(dst1, idn, deg_out, ones_v, idxb, rows16, degacc, sem)


def _agg_body(hpre, srcg2, dst2, idn, out, acc, srcb0, srcb1, dstb0, dstb1,
              idxb, rows0, rows1, sem, gs0, gs1):
    c = lax.axis_index("c")
    s = lax.axis_index("s")

    srcb = (srcb0, srcb1)
    dstb = (dstb0, dstb1)
    rows = (rows0, rows1)
    gs = (gs0, gs1)
    ir = rows0.at[pl.ds(0, IW)]

    def ik(k, carry):
        kk = s * ICT + k
        off = jnp.where(kk < OCT, kk * IW, 0)
        pltpu.sync_copy(idn.at[pl.ds(kk * IW, IW)], idxb)
        pltpu.sync_copy(hpre.at[pl.ds(c * NN + off, IW)], ir)
        pltpu.sync_copy(ir, acc.at[idxb])
        return carry
    lax.fori_loop(0, ICT, ik, 0)
    plsc.subcore_barrier()
    soff = (c * NS + s) * EPW * EW
    doff = s * EPW * EW

    def fetch(g, b):
        pltpu.sync_copy(srcg2.at[pl.ds(soff + g * EW, EW)], srcb[b])
        pltpu.sync_copy(dst2.at[pl.ds(doff + g * EW, EW)], dstb[b])
        pltpu.async_copy(hpre.at[srcb[b]], rows[b], gs[b])

    def drain(b):
        pltpu.make_async_copy(hpre.at[srcb[b]], rows[b], gs[b]).wait()

    fetch(0, 0)
    fetch(1, 1)

    def step(h, carry):
        for b in range(2):
            g = 2 * h + b
            drain(b)
            pltpu.sync_copy(rows[b], acc.at[dstb[b]], add=True)
            nxt = jnp.where(g + 2 < EPW, g + 2, 0)
            fetch(nxt, b)
        return carry
    lax.fori_loop(0, EPW // 2, step, 0)
    drain(0)
    drain(1)
    plsc.subcore_barrier()

    nk = jnp.where(s < NS - 1, ICT, OCL)

    def ok(k, carry):
        row = (s * ICT + k) * IW
        pltpu.sync_copy(idn.at[pl.ds(row, IW)], idxb)
        pltpu.async_copy(acc.at[idxb], ir, sem).wait()
        pltpu.sync_copy(ir, out.at[pl.ds(c * NN + row, IW)])
        return carry
    lax.fori_loop(0, nk, ok, 0)


@functools.partial(
    pl.kernel,
    out_type=jax.ShapeDtypeStruct((2 * NN, HALF), jnp.float32),
    mesh=_MESH,
    scratch_types=[
        pltpu.VMEM_SHARED((NN + 8, HALF), jnp.float32),
        pltpu.VMEM((EW,), jnp.int32),
        pltpu.VMEM((EW,), jnp.int32),
        pltpu.VMEM((EW,), jnp.int32),
        pltpu.VMEM((EW,), jnp.int32),
        pltpu.VMEM((IW,), jnp.int32),
        pltpu.VMEM((EW, HALF), jnp.float32),
        pltpu.VMEM((EW, HALF), jnp.float32),
        pltpu.SemaphoreType.DMA,
        pltpu.SemaphoreType.DMA,
        pltpu.SemaphoreType.DMA,
    ],
)
def _agg(hpre, srcg2, dst2, idn, out, acc, srcb0, srcb1, dstb0, dstb1,
         idxb, rows0, rows1, sem, gs0, gs1):
    _agg_body(hpre, srcg2, dst2, idn, out, acc, srcb0, srcb1, dstb0, dstb1,
              idxb, rows0, rows1, sem, gs0, gs1)


RB = 1000
NBLK = NN // RB


def _tc0_body(x_ref, deg_ref, w_ref, out_ref):
    dinv = lax.rsqrt(deg_ref[:, 0:1])
    h = jnp.dot(x_ref[:], w_ref[:], preferred_element_type=jnp.float32)
    h = h * dinv
    out_ref[0] = h[:, :HALF]
    out_ref[1] = h[:, HALF:]


def _tc0(x, deg16, w):
    return pl.pallas_call(
        _tc0_body,
        grid=(NBLK,),
        in_specs=[
            pl.BlockSpec((RB, FIN), lambda i: (i, 0)),
            pl.BlockSpec((RB, 16), lambda i: (i, 0)),
            pl.BlockSpec((FIN, DD), lambda i: (0, 0)),
        ],
        out_specs=pl.BlockSpec((2, RB, HALF), lambda i: (0, i, 0)),
        out_shape=jax.ShapeDtypeStruct((2, NN, HALF), jnp.float32),
    )(x, deg16, w)


def _tcm_body(a_ref, deg_ref, w_ref, bprev_ref, out_ref):
    dinv = lax.rsqrt(deg_ref[:, 0:1])
    h = jnp.concatenate([a_ref[0], a_ref[1]], axis=1)
    h = h * dinv + bprev_ref[:]
    h = jnp.where(h > 0, h, 0.01 * h)
    h2 = jnp.dot(h, w_ref[:], preferred_element_type=jnp.float32) * dinv
    out_ref[0] = h2[:, :HALF]
    out_ref[1] = h2[:, HALF:]


def _tcm(a, deg16, w, bprev):
    return pl.pallas_call(
        _tcm_body,
        grid=(NBLK,),
        in_specs=[
            pl.BlockSpec((2, RB, HALF), lambda i: (0, i, 0)),
            pl.BlockSpec((RB, 16), lambda i: (i, 0)),
            pl.BlockSpec((DD, DD), lambda i: (0, 0)),
            pl.BlockSpec((1, DD), lambda i: (0, 0)),
        ],
        out_specs=pl.BlockSpec((2, RB, HALF), lambda i: (0, i, 0)),
        out_shape=jax.ShapeDtypeStruct((2, NN, HALF), jnp.float32),
    )(a, deg16, w, bprev)


def _gene_body(exp_ref, wg_ref, bg_ref, ww_ref, bw_ref, g_ref):
    e = jnp.dot(exp_ref[:], wg_ref[:], preferred_element_type=jnp.float32)
    e = jnp.maximum(e + bg_ref[:], 0.0)
    g_ref[:] = jnp.dot(e, ww_ref[:], preferred_element_type=jnp.float32) \
        + bw_ref[:]


def _gene(expp, wgp, bg, ww, bw):
    return pl.pallas_call(
        _gene_body,
        out_shape=jax.ShapeDtypeStruct((BG, DD), jnp.float32),
    )(expp, wgp, bg, ww, bw)


def _fin_body(a_ref, deg_ref, batch_ref, g_ref, b2_ref, wf1_ref, bf1_ref,
              wf2_ref, bf2_ref, out_ref, seg_acc, cnt_acc):
    i = pl.program_id(0)

    @pl.when(i == 0)
    def _():
        seg_acc[:] = jnp.zeros_like(seg_acc)
        cnt_acc[:] = jnp.zeros_like(cnt_acc)

    dinv = lax.rsqrt(deg_ref[:, 0:1])
    h3 = jnp.concatenate([a_ref[0], a_ref[1]], axis=1) * dinv + b2_ref[:]
    b = batch_ref[:][0, 0]
    oh = (lax.broadcasted_iota(jnp.int32, (BG, RB), 0)
          == b[None, :]).astype(jnp.float32)
    seg_acc[:] += jnp.dot(oh, h3, preferred_element_type=jnp.float32)
    cnt_acc[:] += jnp.sum(oh, axis=1, keepdims=True)

    @pl.when(i == NBLK - 1)
    def _():
        cnt = cnt_acc[:]
        pooled = seg_acc[:] / jnp.maximum(cnt, 1.0) \
            + g_ref[:] * (cnt > 0).astype(jnp.float32)
        z = jnp.dot(pooled, wf1_ref[:], preferred_element_type=jnp.float32)
        z = jnp.maximum(z + bf1_ref[:], 0.0)
        out_ref[:] = jnp.dot(z, wf2_ref[:],
                             preferred_element_type=jnp.float32) + bf2_ref[:]


def _fin(a, deg16, batch, g, b2, wf1, bf1, wf2, bf2):
    return pl.pallas_call(
        _fin_body,
        grid=(NBLK,),
        in_specs=[
            pl.BlockSpec((2, RB, HALF), lambda i: (0, i, 0)),
            pl.BlockSpec((RB, 16), lambda i: (i, 0)),
            pl.BlockSpec((1, 1, RB), lambda i: (i, 0, 0)),
            pl.BlockSpec((BG, DD), lambda i: (0, 0)),
            pl.BlockSpec((1, DD), lambda i: (0, 0)),
            pl.BlockSpec((DD, DD), lambda i: (0, 0)),
            pl.BlockSpec((1, DD), lambda i: (0, 0)),
            pl.BlockSpec((DD, 1), lambda i: (0, 0)),
            pl.BlockSpec((1, 1), lambda i: (0, 0)),
        ],
        out_specs=pl.BlockSpec((BG, 1), lambda i: (0, 0)),
        out_shape=jax.ShapeDtypeStruct((BG, 1), jnp.float32),
        scratch_shapes=[
            pltpu.VMEM((BG, DD), jnp.float32),
            pltpu.VMEM((BG, 1), jnp.float32),
        ],
    )(a, deg16, batch, g, b2, wf1, bf1, wf2, bf2)


def kernel(exp, x, edge_index, edge_attr, batch, W_gene, b_gene, W_wg, b_wg,
           W_g0, b_g0, W_g1, b_g1, W_g2, b_g2, W_fc1, b_fc1, W_fc2, b_fc2):
    src = edge_index[0].astype(jnp.int32)
    dst = edge_index[1].astype(jnp.int32)
    srcp = jnp.concatenate([src, jnp.zeros((EP - EE,), jnp.int32)])
    dstp = jnp.concatenate([dst, jnp.full((EP - EE,), DUMP, jnp.int32)])
    dst2 = dstp
    srcg2 = jnp.concatenate([srcp, srcp + NN])

    idn = jnp.concatenate([jnp.arange(NN, dtype=jnp.int32),
                           jnp.full((IDL - NN,), DUMP, jnp.int32)])
    deg16 = _deg(dst2, idn)

    def _do_agg(h2):
        return _agg(h2, srcg2, dst2, idn)

    h = _tc0(x, deg16, W_g0)
    a = _do_agg(h.reshape(2 * NN, HALF))
    h = _tcm(a.reshape(2, NN, HALF), deg16, W_g1, b_g0.reshape(1, DD))
    a = _do_agg(h.reshape(2 * NN, HALF))
    h = _tcm(a.reshape(2, NN, HALF), deg16, W_g2, b_g1.reshape(1, DD))
    a = _do_agg(h.reshape(2 * NN, HALF))

    expp = jnp.pad(exp, ((0, 0), (0, NGP - exp.shape[1])))
    wgp = jnp.pad(W_gene, ((0, NGP - W_gene.shape[0]), (0, 0)))
    g = _gene(expp, wgp, b_gene.reshape(1, -1), W_wg, b_wg.reshape(1, DD))

    return _fin(a.reshape(2, NN, HALF), deg16,
                batch.astype(jnp.int32).reshape(NBLK, 1, RB), g,
                b_g2.reshape(1, DD), W_fc1, b_fc1.reshape(1, DD), W_fc2,
                b_fc2.reshape(1, 1))
